# baseline (speedup 1.0000x reference)
import functools

import jax
import jax.numpy as jnp
from jax import lax
from jax.experimental import pallas as pl
from jax.experimental.pallas import tpu as pltpu
from jax.experimental.pallas import tpu_sc as plsc

L = 2048
C = 16
P = 2 * L - 1
PPAD = 4096
NC = 2
NS = 16
NW = NC * NS
ROWS = L // NW
WIN = ROWS + L
UNROLL = 8


def _build():
    mesh = plsc.VectorSubcoreMesh(core_axis_name="c", subcore_axis_name="s")

    @functools.partial(
        pl.kernel,
        mesh=mesh,
        out_type=jax.ShapeDtypeStruct((L, L, C), jnp.float32),
        scratch_types=[
            pltpu.VMEM((WIN * C,), jnp.float32),
            pltpu.VMEM((WIN, C), jnp.float32),
        ],
        compiler_params=pltpu.CompilerParams(use_tc_tiling_on_sc=False),
    )
    def toeplitz_kernel(table_hbm, out_hbm, fwd_v, rev_v):
        wid = lax.axis_index("s") * NC + lax.axis_index("c")
        base = wid * ROWS

        pltpu.sync_copy(table_hbm.at[pl.ds(base * C, WIN * C)], fwd_v)

        def rev_body(t, carry):
            t0 = t * UNROLL
            for u in range(UNROLL):
                src = (WIN - 1 - (t0 + u)) * C
                rev_v[t0 + u, :] = fwd_v[pl.ds(src, C)]
            return carry

        lax.fori_loop(0, WIN // UNROLL, rev_body, 0)

        def emit_row(r, carry):
            pltpu.sync_copy(
                rev_v.at[pl.ds(ROWS - r, L)],
                out_hbm.at[base + r],
            )
            return carry

        lax.fori_loop(0, ROWS, emit_row, 0)

    return toeplitz_kernel


_KERNEL = _build()


def kernel(params, indices):
    del indices
    table = jnp.concatenate(
        [params[0].reshape(-1), jnp.zeros((PPAD - P) * C, jnp.float32)]
    )
    return _KERNEL(table)


# speedup vs baseline: 3.1308x; 3.1308x over previous
import functools

import jax
import jax.numpy as jnp
from jax import lax
from jax.experimental import pallas as pl
from jax.experimental.pallas import tpu as pltpu
from jax.experimental.pallas import tpu_sc as plsc

L = 2048
C = 16
P = 2 * L - 1
Q = 4096
NC = 2
NS = 16
NW = NC * NS
ROWS = L // NW
CT = C // 8
JT = L // 128
WIN = ROWS + L


def _build():
    mesh = plsc.VectorSubcoreMesh(core_axis_name="c", subcore_axis_name="s")

    @functools.partial(
        pl.kernel,
        mesh=mesh,
        out_type=jax.ShapeDtypeStruct((L, CT, JT, 8, 128), jnp.float32),
        scratch_types=[
            pltpu.VMEM((C, WIN), jnp.float32),
            pltpu.VMEM((CT, JT, 8, 128), jnp.float32),
            pltpu.VMEM((CT, JT, 8, 128), jnp.float32),
            pltpu.SemaphoreType.DMA,
        ],
        compiler_params=pltpu.CompilerParams(use_tc_tiling_on_sc=False),
    )
    def toeplitz_kernel(table_hbm, out_hbm, w_v, pa_v, pb_v, sem):
        wid = lax.axis_index("s") * NC + lax.axis_index("c")
        base = wid * ROWS

        pltpu.sync_copy(table_hbm.at[:, pl.ds(base, WIN)], w_v)

        def rev_body(n, carry):
            for u in range(4):
                k = n * 4 + u
                c = k // 66
                m = k - c * 66
                a = 16 * m
                b = WIN - 16 * m - 16
                va = w_v[c, pl.ds(a, 16)]
                vb = w_v[c, pl.ds(b, 16)]
                w_v[c, pl.ds(a, 16)] = jnp.flip(vb)
                w_v[c, pl.ds(b, 16)] = jnp.flip(va)
            return carry

        lax.fori_loop(0, C * (WIN // 32) // 4, rev_body, 0)

        def build_plane(r, plane):
            s0 = ROWS - r

            def jt_body(jt, carry):
                s = s0 + jt * 128
                for ct in range(CT):
                    for cs in range(8):
                        cc = ct * 8 + cs
                        for jv in range(8):
                            plane[ct, jt, cs, pl.ds(jv * 16, 16)] = (
                                w_v[cc, pl.ds(s + jv * 16, 16)]
                            )
                return carry

            lax.fori_loop(0, JT, jt_body, 0)

        def emit_pair(g, carry):
            ra = 2 * g
            rb = ra + 1
            build_plane(ra, pa_v)
            ca = pltpu.async_copy(pa_v, out_hbm.at[base + ra], sem)
            build_plane(rb, pb_v)
            cb = pltpu.async_copy(pb_v, out_hbm.at[base + rb], sem)
            ca.wait()
            cb.wait()
            return carry

        lax.fori_loop(0, ROWS // 2, emit_pair, 0)

    return toeplitz_kernel


_KERNEL = _build()


def kernel(params, indices):
    del indices
    tab = jnp.concatenate(
        [params[0].T, jnp.zeros((C, Q - P), jnp.float32)], axis=1
    )
    out5 = _KERNEL(tab)
    return out5.transpose(0, 2, 4, 1, 3).reshape(L, L, C)


# speedup vs baseline: 13.6744x; 4.3677x over previous
import functools

import jax
import jax.numpy as jnp
from jax import lax
from jax.experimental import pallas as pl
from jax.experimental.pallas import tpu as pltpu
from jax.experimental.pallas import tpu_sc as plsc

L = 2048
C = 16
P = 2 * L - 1
PAD0 = 8
QQ = 4104
NC = 2
NS = 16
NW = NC * NS
ROWS = L // NW
CT = C // 8
JT = L // 128
FW = 2568
VW = 2560


def _build():
    mesh = plsc.VectorSubcoreMesh(core_axis_name="c", subcore_axis_name="s")

    @functools.partial(
        pl.kernel,
        mesh=mesh,
        out_type=jax.ShapeDtypeStruct((L, CT, JT, 8, 128), jnp.float32),
        scratch_types=[
            pltpu.VMEM((C, FW), jnp.float32),
            pltpu.VMEM((C, VW), jnp.float32),
            pltpu.SemaphoreType.DMA,
        ],
        compiler_params=pltpu.CompilerParams(use_tc_tiling_on_sc=False),
    )
    def toeplitz_kernel(table_hbm, out_hbm, fw_v, v_v, sem):
        wid = lax.axis_index("s") * NC + lax.axis_index("c")
        d = wid & 7
        lo = (wid >> 3) * 512

        pltpu.sync_copy(table_hbm.at[:, pl.ds(lo, FW)], fw_v)

        def rev_body(n, carry):
            for u in range(4):
                k = n * 4 + u
                c = k // (VW // 16)
                m = k - c * (VW // 16)
                src = d + 2544 - 16 * m
                v_v[c, pl.ds(16 * m, 16)] = jnp.flip(fw_v[c, pl.ds(src, 16)])
            return carry

        lax.fori_loop(0, C * (VW // 16) // 4, rev_body, 0)

        def emit_plane(k, carry):
            i = lo + d + 8 * k
            y0 = 504 - 8 * k
            copies = []
            for ct in range(CT):
                for jt in range(JT):
                    copies.append(pltpu.async_copy(
                        v_v.at[pl.ds(ct * 8, 8), pl.ds(y0 + jt * 128, 128)],
                        out_hbm.at[i, ct, jt],
                        sem,
                    ))
            for cp in copies:
                cp.wait()
            return carry

        lax.fori_loop(0, ROWS, emit_plane, 0)

    return toeplitz_kernel


_KERNEL = _build()


def kernel(params, indices):
    del indices
    tab = jnp.concatenate(
        [
            jnp.zeros((C, PAD0), jnp.float32),
            params[0].T,
            jnp.zeros((C, QQ - PAD0 - P), jnp.float32),
        ],
        axis=1,
    )
    out5 = _KERNEL(tab)
    return out5.transpose(0, 2, 4, 1, 3).reshape(L, L, C)
